# Initial kernel scaffold; baseline (speedup 1.0000x reference)
import jax
import jax.numpy as jnp
from jax.experimental import pallas as pl


def kernel(embedding, to_children_edges, to_ancestor_edges, word_indexes, W, att_src, att_dst, bias):
    raise NotImplementedError("write your pallas kernel here")


# speedup vs baseline: 95.5846x; 95.5846x over previous
import functools

import jax
import jax.numpy as jnp
from jax import lax
from jax.experimental import pallas as pl
from jax.experimental.pallas import tpu as pltpu
from jax.experimental.pallas import tpu_sc as plsc

N = 10000
V = 8000
E = 320000
IN = 128
HEADS = 8
OUT = 16
HC = HEADS * OUT
NEG = 0.2

NC, NS = 2, 16
NW = NC * NS
B = 128
NCHUNK = E // B
CPT = -(-NCHUNK // NW)

R_BIG = 640
R_SMALL = N - 15 * R_BIG
VPAD = 8192
VPW = VPAD // NW

_f32 = jnp.float32


def _proj_body(x_ref, w_ref, as_ref, ad_ref, h_ref, s_ref, d_ref):
    x = x_ref[...]
    h = lax.dot_general(x, w_ref[...], (((1,), (1,)), ((), ())),
                        preferred_element_type=_f32)
    h_ref[...] = h
    s_ref[...] = lax.dot_general(h, as_ref[...], (((1,), (0,)), ((), ())),
                                 preferred_element_type=_f32)
    d_ref[...] = lax.dot_general(h, ad_ref[...], (((1,), (0,)), ((), ())),
                                 preferred_element_type=_f32)


def _norm_proj_body(acc_ref, den_ref, b16_ref, bias_ref, w_ref, as_ref,
                    ad_ref, h_ref, s_ref, d_ref):
    acc = acc_ref[0] + acc_ref[1]
    den = den_ref[0] + den_ref[1]
    denb = lax.dot_general(den, b16_ref[...], (((1,), (0,)), ((), ())),
                           preferred_element_type=_f32)
    x = acc / (denb + 1e-16) + bias_ref[...]
    h = lax.dot_general(x, w_ref[...], (((1,), (1,)), ((), ())),
                        preferred_element_type=_f32)
    h_ref[...] = h
    s_ref[...] = lax.dot_general(h, as_ref[...], (((1,), (0,)), ((), ())),
                                 preferred_element_type=_f32)
    d_ref[...] = lax.dot_general(h, ad_ref[...], (((1,), (0,)), ((), ())),
                                 preferred_element_type=_f32)


def _norm_body(acc_ref, den_ref, b16_ref, bias_ref, x_ref):
    acc = acc_ref[0] + acc_ref[1]
    den = den_ref[0] + den_ref[1]
    denb = lax.dot_general(den, b16_ref[...], (((1,), (0,)), ((), ())),
                           preferred_element_type=_f32)
    x_ref[...] = acc / (denb + 1e-16) + bias_ref[...]


_R = 1000


def _tc_proj(x, w, asel, adsel):
    return pl.pallas_call(
        _proj_body,
        grid=(N // _R,),
        in_specs=[
            pl.BlockSpec((_R, IN), lambda i: (i, 0)),
            pl.BlockSpec((HC, IN), lambda i: (0, 0)),
            pl.BlockSpec((HC, 16), lambda i: (0, 0)),
            pl.BlockSpec((HC, 16), lambda i: (0, 0)),
        ],
        out_specs=[
            pl.BlockSpec((_R, HC), lambda i: (i, 0)),
            pl.BlockSpec((_R, 16), lambda i: (i, 0)),
            pl.BlockSpec((_R, 16), lambda i: (i, 0)),
        ],
        out_shape=[
            jax.ShapeDtypeStruct((N, HC), _f32),
            jax.ShapeDtypeStruct((N, 16), _f32),
            jax.ShapeDtypeStruct((N, 16), _f32),
        ],
    )(x, w, asel, adsel)


def _tc_norm_proj(accp, denp, b16, bias, w, asel, adsel):
    return pl.pallas_call(
        _norm_proj_body,
        grid=(N // _R,),
        in_specs=[
            pl.BlockSpec((2, _R, HC), lambda i: (0, i, 0)),
            pl.BlockSpec((2, _R, 16), lambda i: (0, i, 0)),
            pl.BlockSpec((16, HC), lambda i: (0, 0)),
            pl.BlockSpec((HC,), lambda i: (0,)),
            pl.BlockSpec((HC, IN), lambda i: (0, 0)),
            pl.BlockSpec((HC, 16), lambda i: (0, 0)),
            pl.BlockSpec((HC, 16), lambda i: (0, 0)),
        ],
        out_specs=[
            pl.BlockSpec((_R, HC), lambda i: (i, 0)),
            pl.BlockSpec((_R, 16), lambda i: (i, 0)),
            pl.BlockSpec((_R, 16), lambda i: (i, 0)),
        ],
        out_shape=[
            jax.ShapeDtypeStruct((N, HC), _f32),
            jax.ShapeDtypeStruct((N, 16), _f32),
            jax.ShapeDtypeStruct((N, 16), _f32),
        ],
    )(accp, denp, b16, bias, w, asel, adsel)


def _tc_norm(accp, denp, b16, bias):
    return pl.pallas_call(
        _norm_body,
        grid=(N // _R,),
        in_specs=[
            pl.BlockSpec((2, _R, HC), lambda i: (0, i, 0)),
            pl.BlockSpec((2, _R, 16), lambda i: (0, i, 0)),
            pl.BlockSpec((16, HC), lambda i: (0, 0)),
            pl.BlockSpec((HC,), lambda i: (0,)),
        ],
        out_specs=pl.BlockSpec((_R, HC), lambda i: (i, 0)),
        out_shape=jax.ShapeDtypeStruct((N, HC), _f32),
    )(accp, denp, b16, bias)


_MESH = dict(core_axis_name="c", subcore_axis_name="s", num_cores=NC,
             num_subcores=NS)


def _edge_body(src_hbm, dst_hbm, s_hbm, d_hbm, h_hbm, accp_hbm, denp_hbm,
               sidx, didx, srow, drow, prow, rows, acc_sh, den_sh,
               sem1, sem2, sem3):
    cid = lax.axis_index("c")
    sid = lax.axis_index("s")
    w = sid * NC + cid

    zero16 = jnp.zeros((16,), _f32)

    @pl.loop(0, B)
    def _(b):
        prow[b, :] = zero16
        for j in range(HEADS):
            rows[b, pl.ds(16 * j, 16)] = zero16

    r0 = sid * R_BIG
    n_z = jnp.where(sid < 15, R_BIG // 80, R_SMALL // 80)

    @pl.loop(0, n_z)
    def _(t):
        pltpu.sync_copy(rows.at[pl.ds(0, 80)],
                        acc_sh.at[pl.ds(r0 + t * 80, 80)])
        pltpu.sync_copy(prow.at[pl.ds(0, 80)],
                        den_sh.at[pl.ds(r0 + t * 80, 80)])

    plsc.subcore_barrier()

    @pl.loop(0, CPT)
    def _(t):
        c = t * NW + w

        @pl.when(c < NCHUNK)
        def _():
            base = c * B
            pltpu.sync_copy(src_hbm.at[pl.ds(base, B)], sidx)
            pltpu.sync_copy(dst_hbm.at[pl.ds(base, B)], didx)
            cp1 = pltpu.async_copy(s_hbm.at[sidx], srow, sem1)
            cp2 = pltpu.async_copy(d_hbm.at[didx], drow, sem2)
            cp3 = pltpu.async_copy(h_hbm.at[sidx], rows, sem3)
            cp1.wait()
            cp2.wait()

            @pl.loop(0, B)
            def _(b):
                a = srow[b, :] + drow[b, :]
                a = jnp.where(a > 0, a, a * NEG)
                prow[b, :] = jnp.exp(a)

            pltpu.sync_copy(prow, den_sh.at[didx], add=True)
            cp3.wait()

            @pl.loop(0, B)
            def _(b):
                pv = prow[b, :]
                for j in range(HEADS):
                    sc = jnp.full((16,), pv[j], _f32)
                    rows[b, pl.ds(16 * j, 16)] = rows[b, pl.ds(16 * j, 16)] * sc

            pltpu.sync_copy(rows, acc_sh.at[didx], add=True)

    plsc.subcore_barrier()

    @pl.when(sid < 15)
    def _():
        pltpu.sync_copy(acc_sh.at[pl.ds(r0, R_BIG)],
                        accp_hbm.at[cid, pl.ds(r0, R_BIG)])
        pltpu.sync_copy(den_sh.at[pl.ds(r0, R_BIG)],
                        denp_hbm.at[cid, pl.ds(r0, R_BIG)])

    @pl.when(sid == 15)
    def _():
        pltpu.sync_copy(acc_sh.at[pl.ds(15 * R_BIG, R_SMALL)],
                        accp_hbm.at[cid, pl.ds(15 * R_BIG, R_SMALL)])
        pltpu.sync_copy(den_sh.at[pl.ds(15 * R_BIG, R_SMALL)],
                        denp_hbm.at[cid, pl.ds(15 * R_BIG, R_SMALL)])


def _sc_edge(src, dst, s_tab, d_tab, h_tab):
    k = pl.kernel(
        _edge_body,
        out_type=[
            jax.ShapeDtypeStruct((NC, N, HC), _f32),
            jax.ShapeDtypeStruct((NC, N, 16), _f32),
        ],
        mesh=plsc.VectorSubcoreMesh(**_MESH),
        compiler_params=pltpu.CompilerParams(use_tc_tiling_on_sc=False),
        scratch_types=[
            pltpu.VMEM((B,), jnp.int32),
            pltpu.VMEM((B,), jnp.int32),
            pltpu.VMEM((B, 16), _f32),
            pltpu.VMEM((B, 16), _f32),
            pltpu.VMEM((B, 16), _f32),
            pltpu.VMEM((B, HC), _f32),
            pltpu.VMEM_SHARED((N, HC), _f32),
            pltpu.VMEM_SHARED((N, 16), _f32),
            pltpu.SemaphoreType.DMA,
            pltpu.SemaphoreType.DMA,
            pltpu.SemaphoreType.DMA,
        ],
    )
    return k(src, dst, s_tab, d_tab, h_tab)


def _gather_body(x_hbm, wi_hbm, out_hbm, idx_v, rows_v, sem):
    wid = lax.axis_index("s") * NC + lax.axis_index("c")
    base = wid * VPW
    pltpu.sync_copy(wi_hbm.at[pl.ds(base, VPW)], idx_v)
    pltpu.async_copy(x_hbm.at[idx_v], rows_v, sem).wait()
    pltpu.sync_copy(rows_v, out_hbm.at[pl.ds(base, VPW)])


def _sc_gather(x, wi_pad):
    k = pl.kernel(
        _gather_body,
        out_type=jax.ShapeDtypeStruct((VPAD, HC), _f32),
        mesh=plsc.VectorSubcoreMesh(**_MESH),
        scratch_types=[
            pltpu.VMEM((VPW,), jnp.int32),
            pltpu.VMEM((VPW, HC), _f32),
            pltpu.SemaphoreType.DMA,
        ],
    )
    return k(x, wi_pad)


def kernel(embedding, to_children_edges, to_ancestor_edges, word_indexes, W,
           att_src, att_dst, bias):
    j = jnp.arange(HC)
    hp = jnp.arange(16)
    sel = (j[:, None] // OUT == hp[None, :] % HEADS).astype(_f32)
    asel = att_src.reshape(HC, 1) * sel
    adsel = att_dst.reshape(HC, 1) * sel
    b16 = ((hp[:, None] == j[None, :] // OUT) & (hp[:, None] < HEADS))
    b16 = b16.astype(_f32)

    src1, dst1 = to_children_edges[0], to_children_edges[1]
    src2, dst2 = to_ancestor_edges[0], to_ancestor_edges[1]
    wi_pad = jnp.concatenate(
        [word_indexes, jnp.zeros((VPAD - V,), jnp.int32)])

    h1, s1, d1 = _tc_proj(embedding, W, asel, adsel)
    accp1, denp1 = _sc_edge(src1, dst1, s1, d1, h1)
    h2, s2, d2 = _tc_norm_proj(accp1, denp1, b16, bias, W, asel, adsel)
    accp2, denp2 = _sc_edge(src2, dst2, s2, d2, h2)
    x3 = _tc_norm(accp2, denp2, b16, bias)
    out = _sc_gather(x3, wi_pad)
    return out[:V]
